# Initial kernel scaffold; baseline (speedup 1.0000x reference)
import jax
import jax.numpy as jnp
from jax.experimental import pallas as pl


def kernel(x, edge_index, edge_attr, batch, W_emb, b_emb, W_rel1, b_rel1, W_root1, gamma1, beta1, W_rel2, b_rel2, W_root2, gamma2, beta2, Wc1, bc1, Wc2, bc2):
    raise NotImplementedError("write your pallas kernel here")


# speedup vs baseline: 9.3889x; 9.3889x over previous
import functools

import jax
import jax.numpy as jnp
from jax import lax
from jax.experimental import pallas as pl
from jax.experimental.pallas import tpu as pltpu
from jax.experimental.pallas import tpu_sc as plsc

_N_NODES = 10000
_N_EDGES = 320000
_HID = 32
_NUM_GRAPHS = 64

_NC = 2
_NS = 16
_SUB = 128
_NW = _NC * _NS
_WPW = 80
_E_PAD = _NW * _WPW * _SUB
_NCH = _N_NODES // 8
_NCH_BASE = _NCH // _NS
_NCH_EXTRA = _NCH - _NCH_BASE * _NS


def _mp_body(g_hbm, src_hbm, dst_hbm, ew_hbm, zeros_hbm, out_hbm,
             g_sp, agg_sp, srcv, dstv, eww, rows, sem):
    cid = lax.axis_index("c")
    sid = lax.axis_index("s")
    gwid = cid * _NS + sid

    r0 = 8 * (_NCH_BASE * sid + jnp.minimum(sid, _NCH_EXTRA))
    nbase = 8 * _NCH_BASE
    pltpu.sync_copy(g_hbm.at[pl.ds(r0, nbase)], g_sp.at[pl.ds(r0, nbase)])
    pltpu.sync_copy(zeros_hbm.at[pl.ds(r0, nbase)], agg_sp.at[pl.ds(r0, nbase)])

    @pl.when(sid < _NCH_EXTRA)
    def _():
        pltpu.sync_copy(g_hbm.at[pl.ds(r0 + nbase, 8)],
                        g_sp.at[pl.ds(r0 + nbase, 8)])
        pltpu.sync_copy(zeros_hbm.at[pl.ds(r0 + nbase, 8)],
                        agg_sp.at[pl.ds(r0 + nbase, 8)])

    pltpu.sync_copy(src_hbm.at[gwid], srcv)
    pltpu.sync_copy(dst_hbm.at[gwid], dstv)
    pltpu.sync_copy(ew_hbm.at[gwid], eww)

    plsc.subcore_barrier()

    def window(j, carry):
        pltpu.async_copy(g_sp.at[srcv.at[j]], rows, sem).wait()
        for g in range(_SUB // 16):
            ewv = eww[j, pl.ds(g * 16, 16)]
            for e in range(16):
                w = ewv[e]
                r = g * 16 + e
                rows[r, pl.ds(0, 16)] = rows[r, pl.ds(0, 16)] * w
                rows[r, pl.ds(16, 16)] = rows[r, pl.ds(16, 16)] * w
        pltpu.sync_copy(rows, agg_sp.at[dstv.at[j]], add=True)
        return carry

    lax.fori_loop(0, _WPW, window, 0)

    plsc.subcore_barrier()
    pltpu.sync_copy(agg_sp.at[pl.ds(r0, nbase)],
                    out_hbm.at[cid, pl.ds(r0, nbase)])

    @pl.when(sid < _NCH_EXTRA)
    def _():
        pltpu.sync_copy(agg_sp.at[pl.ds(r0 + nbase, 8)],
                        out_hbm.at[cid, pl.ds(r0 + nbase, 8)])


def _message_pass(g, src2d, dst2d, ew2d, zeros):
    mesh = plsc.VectorSubcoreMesh(core_axis_name="c", subcore_axis_name="s",
                                  num_cores=_NC, num_subcores=_NS)
    run = pl.kernel(
        _mp_body,
        out_type=jax.ShapeDtypeStruct((_NC, _N_NODES, _HID), jnp.float32),
        mesh=mesh,
        scratch_types=[
            pltpu.VMEM_SHARED((_N_NODES, _HID), jnp.float32),
            pltpu.VMEM_SHARED((_N_NODES, _HID), jnp.float32),
            pltpu.VMEM((_WPW, _SUB), jnp.int32),
            pltpu.VMEM((_WPW, _SUB), jnp.int32),
            pltpu.VMEM((_WPW, _SUB), jnp.float32),
            pltpu.VMEM((_SUB, _HID), jnp.float32),
            pltpu.SemaphoreType.DMA,
        ],
    )
    return run(g, src2d, dst2d, ew2d, zeros)


def _dense_a_body(x_ref, we_ref, be_ref, wr_ref, h_ref, g_ref):
    h = jnp.dot(x_ref[...], we_ref[...],
                preferred_element_type=jnp.float32) + be_ref[...]
    h_ref[...] = h
    g_ref[...] = jnp.dot(h, wr_ref[...], preferred_element_type=jnp.float32)


def _dense_b_body(p_ref, h_ref, wroot_ref, brel_ref, gamma_ref, beta_ref,
                  wrel2_ref, h1_ref, g2_ref):
    conv = (p_ref[0] + p_ref[1] + brel_ref[...] +
            jnp.dot(h_ref[...], wroot_ref[...],
                    preferred_element_type=jnp.float32))
    mean = jnp.mean(conv, axis=0, keepdims=True)
    var = jnp.mean(jnp.square(conv - mean), axis=0, keepdims=True)
    h1 = jnp.maximum(
        (conv - mean) / jnp.sqrt(var + 1e-5) * gamma_ref[...] + beta_ref[...],
        0.0)
    h1_ref[...] = h1
    g2_ref[...] = jnp.dot(h1, wrel2_ref[...], preferred_element_type=jnp.float32)


def _dense_c_body(p_ref, h_ref, wroot_ref, brel_ref, gamma_ref, beta_ref,
                  batch_ref, wc1_ref, bc1_ref, wc2_ref, bc2_ref,
                  out_ref, gx_ref):
    conv = (p_ref[0] + p_ref[1] + brel_ref[...] +
            jnp.dot(h_ref[...], wroot_ref[...],
                    preferred_element_type=jnp.float32))
    mean = jnp.mean(conv, axis=0, keepdims=True)
    var = jnp.mean(jnp.square(conv - mean), axis=0, keepdims=True)
    h2 = jnp.maximum(
        (conv - mean) / jnp.sqrt(var + 1e-5) * gamma_ref[...] + beta_ref[...],
        0.0)
    batch = batch_ref[...]

    def seg(gid, carry):
        m = batch == gid
        gx_ref[pl.ds(gid, 1), :] = jnp.max(
            jnp.where(m, h2, -jnp.inf), axis=0, keepdims=True)
        return carry

    lax.fori_loop(0, _NUM_GRAPHS, seg, 0)
    hidc = jnp.maximum(
        jnp.dot(gx_ref[...], wc1_ref[...],
                preferred_element_type=jnp.float32) + bc1_ref[...], 0.0)
    out_ref[...] = jnp.dot(hidc, wc2_ref[...],
                           preferred_element_type=jnp.float32) + bc2_ref[...]


def kernel(x, edge_index, edge_attr, batch, W_emb, b_emb, W_rel1, b_rel1,
           W_root1, gamma1, beta1, W_rel2, b_rel2, W_root2, gamma2, beta2,
           Wc1, bc1, Wc2, bc2):
    pad = _E_PAD - _N_EDGES
    src2d = jnp.concatenate(
        [edge_index[0], jnp.zeros((pad,), jnp.int32)]).reshape(_NW, _WPW, _SUB)
    dst2d = jnp.concatenate(
        [edge_index[1], jnp.zeros((pad,), jnp.int32)]).reshape(_NW, _WPW, _SUB)
    ew2d = jnp.concatenate(
        [edge_attr.reshape(-1), jnp.zeros((pad,), jnp.float32)]
    ).reshape(_NW, _WPW, _SUB)
    zeros = jnp.zeros((_N_NODES, _HID), jnp.float32)
    batch2d = batch.reshape(_N_NODES, 1)

    h0, g1 = pl.pallas_call(
        _dense_a_body,
        out_shape=(jax.ShapeDtypeStruct((_N_NODES, _HID), jnp.float32),
                   jax.ShapeDtypeStruct((_N_NODES, _HID), jnp.float32)),
    )(x, W_emb, b_emb.reshape(1, _HID), W_rel1)

    p1 = _message_pass(g1, src2d, dst2d, ew2d, zeros)

    h1, g2 = pl.pallas_call(
        _dense_b_body,
        out_shape=(jax.ShapeDtypeStruct((_N_NODES, _HID), jnp.float32),
                   jax.ShapeDtypeStruct((_N_NODES, _HID), jnp.float32)),
    )(p1, h0, W_root1, b_rel1.reshape(1, _HID), gamma1.reshape(1, _HID),
      beta1.reshape(1, _HID), W_rel2)

    p2 = _message_pass(g2, src2d, dst2d, ew2d, zeros)

    out, _ = pl.pallas_call(
        _dense_c_body,
        out_shape=(jax.ShapeDtypeStruct((_NUM_GRAPHS, 10), jnp.float32),
                   jax.ShapeDtypeStruct((_NUM_GRAPHS, _HID), jnp.float32)),
    )(p2, h1, W_root2, b_rel2.reshape(1, _HID), gamma2.reshape(1, _HID),
      beta2.reshape(1, _HID), batch2d, Wc1, bc1.reshape(1, 2 * _HID),
      Wc2, bc2.reshape(1, 10))
    return out
